# baseline (speedup 1.0000x reference)
import functools

import jax
import jax.numpy as jnp
from jax import lax
from jax.experimental import pallas as pl
from jax.experimental.pallas import tpu as pltpu
from jax.experimental.pallas import tpu_sc as plsc

E = 320000
NA = 10000
NM = 512
AF = 133
BF = 147
H = 300
HP = 384
NC = 2
NSUB = 16
GW = 128
NG = HP // GW
QW = 80
CHUNK = 80
NBUF = 4
CPS = (E // CHUNK) // NSUB
CPS -= CPS % NBUF
NXTRA = (E // CHUNK) - NSUB * CPS
RPS = 624
NREM = NA - NSUB * RPS
ZR = 16
STR = 26
TE = 512
GE = E // TE
TA = 1000
GA = NA // TA
TASK = 2

_f32 = jnp.float32


def _zero_zbuf(zbuf, width):
    zv = jnp.zeros((16,), _f32)
    for r in range(ZR):
        for k in range(width // 16):
            zbuf[r, pl.ds(k * 16, 16)] = zv


def _zero_rows(zbuf, acc, s):
    r0 = s * RPS
    for j in range(RPS // ZR):
        pltpu.sync_copy(zbuf, acc.at[pl.ds(r0 + j * ZR, ZR)])

    @pl.when(s == NSUB - 1)
    def _():
        pltpu.sync_copy(zbuf.at[pl.ds(0, NREM)],
                        acc.at[pl.ds(NSUB * RPS, NREM)])


def _sc_seg_body(do_expand, flip, h_hbm, src_hbm, dst_hbm, z_hbm, out_hbm,
                 acc, b0, b1, b2, b3, i0b, i1b, i2b, i3b, zbuf,
                 l0, l1, l2, l3, t0, t1, t2, t3):
    c = lax.axis_index("c")
    if flip:
        c = 1 - c
    s = lax.axis_index("s")
    bufs = (b0, b1, b2, b3)
    idxs = (i0b, i1b, i2b, i3b)
    lsem = (l0, l1, l2, l3)
    tsem = (t0, t1, t2, t3)
    pltpu.sync_copy(z_hbm, zbuf)
    c0 = s * CPS
    nq = CPS // NBUF

    for p in range(2):
        g = c + 2 * p
        active = g < NG
        col0 = pl.multiple_of(g * GW, GW)
        _zero_rows(zbuf, acc, s)
        plsc.subcore_barrier()

        def ld1(i, b):
            pltpu.async_copy(dst_hbm.at[pl.ds(i * CHUNK, CHUNK)],
                             idxs[b], lsem[b])
            pltpu.async_copy(h_hbm.at[pl.ds(i * CHUNK, CHUNK),
                                      pl.ds(col0, GW)],
                             bufs[b], lsem[b])

        def ld1_wait(i, b):
            pltpu.make_async_copy(dst_hbm.at[pl.ds(i * CHUNK, CHUNK)],
                                  idxs[b], lsem[b]).wait()
            pltpu.make_async_copy(h_hbm.at[pl.ds(i * CHUNK, CHUNK),
                                           pl.ds(col0, GW)],
                                  bufs[b], lsem[b]).wait()

        @pl.when(g < NG)
        def _phase1():
            for b in range(NBUF):
                ld1(c0 + b, b)

            def quad1(q, carry):
                for b in range(NBUF):
                    ld1_wait(c0 + q * NBUF + b, b)
                    pltpu.async_copy(bufs[b], acc.at[idxs[b]], tsem[b],
                                     add=True)
                for b in range(NBUF):
                    i = c0 + q * NBUF + b
                    pltpu.make_async_copy(bufs[b], acc.at[idxs[b]],
                                          tsem[b]).wait()

                    @pl.when(q < nq - 1)
                    def _(i=i, b=b):
                        ld1(i + NBUF, b)
                return carry

            lax.fori_loop(0, nq, quad1, 0)

            for x in range(NXTRA // NSUB):
                i = NSUB * (CPS + x) + s
                pltpu.sync_copy(dst_hbm.at[pl.ds(i * CHUNK, CHUNK)], i0b)
                pltpu.sync_copy(h_hbm.at[pl.ds(i * CHUNK, CHUNK),
                                         pl.ds(col0, GW)], b0)
                pltpu.sync_copy(b0, acc.at[i0b], add=True)

            if NXTRA % NSUB:
                @pl.when(s < NXTRA % NSUB)
                def _():
                    i = NSUB * (CPS + NXTRA // NSUB) + s
                    pltpu.sync_copy(dst_hbm.at[pl.ds(i * CHUNK, CHUNK)], i0b)
                    pltpu.sync_copy(h_hbm.at[pl.ds(i * CHUNK, CHUNK),
                                             pl.ds(col0, GW)], b0)
                    pltpu.sync_copy(b0, acc.at[i0b], add=True)

        plsc.subcore_barrier()

        if do_expand:
            def ld2(i, b):
                pltpu.async_copy(src_hbm.at[pl.ds(i * CHUNK, CHUNK)],
                                 idxs[b], lsem[b])

            def ld2_wait(i, b):
                pltpu.make_async_copy(src_hbm.at[pl.ds(i * CHUNK, CHUNK)],
                                      idxs[b], lsem[b]).wait()

            def st2(i, b):
                return pltpu.async_copy(
                    bufs[b],
                    out_hbm.at[pl.ds(i * CHUNK, CHUNK), pl.ds(col0, GW)],
                    lsem[b])

            def st2_wait(i, b):
                pltpu.make_async_copy(
                    bufs[b],
                    out_hbm.at[pl.ds(i * CHUNK, CHUNK), pl.ds(col0, GW)],
                    lsem[b]).wait()

            @pl.when(g < NG)
            def _phase2():
                for b in range(NBUF):
                    ld2(c0 + b, b)

                def quad2(q, carry):
                    for b in range(NBUF):
                        ld2_wait(c0 + q * NBUF + b, b)
                        pltpu.async_copy(acc.at[idxs[b]], bufs[b], tsem[b])
                    for b in range(NBUF):
                        i = c0 + q * NBUF + b
                        pltpu.make_async_copy(acc.at[idxs[b]], bufs[b],
                                              tsem[b]).wait()
                        st2(i, b)
                    for b in range(NBUF):
                        i = c0 + q * NBUF + b
                        st2_wait(i, b)

                        @pl.when(q < nq - 1)
                        def _(i=i, b=b):
                            ld2(i + NBUF, b)
                    return carry

                lax.fori_loop(0, nq, quad2, 0)

                for x in range(NXTRA // NSUB):
                    i = NSUB * (CPS + x) + s
                    pltpu.sync_copy(src_hbm.at[pl.ds(i * CHUNK, CHUNK)], i0b)
                    pltpu.sync_copy(acc.at[i0b], b0)
                    pltpu.sync_copy(b0, out_hbm.at[pl.ds(i * CHUNK, CHUNK),
                                                   pl.ds(col0, GW)])

                if NXTRA % NSUB:
                    @pl.when(s < NXTRA % NSUB)
                    def _():
                        i = NSUB * (CPS + NXTRA // NSUB) + s
                        pltpu.sync_copy(
                            src_hbm.at[pl.ds(i * CHUNK, CHUNK)], i0b)
                        pltpu.sync_copy(acc.at[i0b], b0)
                        pltpu.sync_copy(
                            b0, out_hbm.at[pl.ds(i * CHUNK, CHUNK),
                                           pl.ds(col0, GW)])
        else:
            @pl.when(g < NG)
            def _drain():
                r0 = s * RPS
                pltpu.sync_copy(acc.at[pl.ds(r0, RPS)],
                                out_hbm.at[pl.ds(r0, RPS),
                                           pl.ds(col0, GW)])

                @pl.when(s == NSUB - 1)
                def _():
                    pltpu.sync_copy(
                        acc.at[pl.ds(NSUB * RPS, NREM)],
                        out_hbm.at[pl.ds(NSUB * RPS, NREM),
                                   pl.ds(col0, GW)])

        plsc.subcore_barrier()


def _make_seg(do_expand, flip=False):
    out_shape = (E, HP) if do_expand else (NA, HP)
    return pl.kernel(
        functools.partial(_sc_seg_body, do_expand, flip),
        out_type=jax.ShapeDtypeStruct(out_shape, _f32),
        mesh=plsc.VectorSubcoreMesh(core_axis_name="c", subcore_axis_name="s"),
        scratch_types=(
            [pltpu.VMEM_SHARED((NA, GW), _f32)]
            + [pltpu.VMEM((CHUNK, GW), _f32) for _ in range(NBUF)]
            + [pltpu.VMEM((CHUNK,), jnp.int32) for _ in range(NBUF)]
            + [pltpu.VMEM((ZR, GW), _f32)]
            + [pltpu.SemaphoreType.DMA for _ in range(2 * NBUF)]
        ),
    )


def _sc_diff_body(hd_hbm, src_hbm, dst_hbm, out_hbm,
                  stage, acc, b0, b1, b2, b3, is0, is1, is2, is3,
                  id0, id1, id2, id3, zbuf, stbuf,
                  l0, l1, l2, l3, t0, t1, t2, t3):
    c = lax.axis_index("c")
    s = lax.axis_index("s")
    bufs = (b0, b1, b2, b3)
    sidx = (is0, is1, is2, is3)
    didx = (id0, id1, id2, id3)
    lsem = (l0, l1, l2, l3)
    tsem = (t0, t1, t2, t3)
    r0 = s * RPS
    c0 = s * CPS
    nq = CPS // NBUF
    _zero_zbuf(zbuf, QW)
    for p in range(2):
        g = 2 * c + p
        dcol = pl.multiple_of(g * QW, 16)
        for j in range(RPS // STR):
            pltpu.sync_copy(hd_hbm.at[pl.ds(r0 + j * STR, STR),
                                      pl.ds(dcol, QW)], stbuf)
            pltpu.sync_copy(stbuf, stage.at[pl.ds(r0 + j * STR, STR)])

        @pl.when(s == NSUB - 1)
        def _():
            pltpu.sync_copy(hd_hbm.at[pl.ds(NSUB * RPS, NREM),
                                      pl.ds(dcol, QW)],
                            stbuf.at[pl.ds(0, NREM)])
            pltpu.sync_copy(stbuf.at[pl.ds(0, NREM)],
                            stage.at[pl.ds(NSUB * RPS, NREM)])

        _zero_rows(zbuf, acc, s)
        plsc.subcore_barrier()

        def ld(i, b):
            pltpu.async_copy(src_hbm.at[pl.ds(i * CHUNK, CHUNK)],
                             sidx[b], lsem[b])
            pltpu.async_copy(dst_hbm.at[pl.ds(i * CHUNK, CHUNK)],
                             didx[b], lsem[b])

        def ld_wait(i, b):
            pltpu.make_async_copy(src_hbm.at[pl.ds(i * CHUNK, CHUNK)],
                                  sidx[b], lsem[b]).wait()
            pltpu.make_async_copy(dst_hbm.at[pl.ds(i * CHUNK, CHUNK)],
                                  didx[b], lsem[b]).wait()

        for b in range(NBUF):
            ld(c0 + b, b)

        def quad(q, carry):
            for b in range(NBUF):
                i = c0 + q * NBUF + b
                ld_wait(i, b)
                pltpu.async_copy(stage.at[sidx[b]], bufs[b], tsem[b])
            for b in range(NBUF):
                pltpu.make_async_copy(stage.at[sidx[b]], bufs[b],
                                      tsem[b]).wait()
                pltpu.async_copy(bufs[b], acc.at[didx[b]], tsem[b], add=True)
            for b in range(NBUF):
                i = c0 + q * NBUF + b
                pltpu.make_async_copy(bufs[b], acc.at[didx[b]],
                                      tsem[b]).wait()

                @pl.when(q < nq - 1)
                def _(i=i, b=b):
                    ld(i + NBUF, b)
            return carry

        lax.fori_loop(0, nq, quad, 0)

        for x in range(NXTRA // NSUB):
            i = NSUB * (CPS + x) + s
            pltpu.sync_copy(src_hbm.at[pl.ds(i * CHUNK, CHUNK)], is0)
            pltpu.sync_copy(dst_hbm.at[pl.ds(i * CHUNK, CHUNK)], id0)
            pltpu.sync_copy(stage.at[is0], b0)
            pltpu.sync_copy(b0, acc.at[id0], add=True)

        if NXTRA % NSUB:
            @pl.when(s < NXTRA % NSUB)
            def _():
                i = NSUB * (CPS + NXTRA // NSUB) + s
                pltpu.sync_copy(src_hbm.at[pl.ds(i * CHUNK, CHUNK)], is0)
                pltpu.sync_copy(dst_hbm.at[pl.ds(i * CHUNK, CHUNK)], id0)
                pltpu.sync_copy(stage.at[is0], b0)
                pltpu.sync_copy(b0, acc.at[id0], add=True)

        plsc.subcore_barrier()
        pltpu.sync_copy(acc.at[pl.ds(r0, RPS)],
                        out_hbm.at[g, pl.ds(r0, RPS)])

        @pl.when(s == NSUB - 1)
        def _():
            pltpu.sync_copy(acc.at[pl.ds(NSUB * RPS, NREM)],
                            out_hbm.at[g, pl.ds(NSUB * RPS, NREM)])

        plsc.subcore_barrier()


_sc_diff = pl.kernel(
    _sc_diff_body,
    out_type=jax.ShapeDtypeStruct((4, NA, QW), _f32),
    mesh=plsc.VectorSubcoreMesh(core_axis_name="c", subcore_axis_name="s"),
    compiler_params=pltpu.CompilerParams(use_tc_tiling_on_sc=False),
    scratch_types=(
        [pltpu.VMEM_SHARED((NA, QW), _f32),
         pltpu.VMEM_SHARED((NA, QW), _f32)]
        + [pltpu.VMEM((CHUNK, QW), _f32) for _ in range(NBUF)]
        + [pltpu.VMEM((CHUNK,), jnp.int32) for _ in range(2 * NBUF)]
        + [pltpu.VMEM((ZR, QW), _f32)]
        + [pltpu.VMEM((STR, QW), _f32)]
        + [pltpu.SemaphoreType.DMA for _ in range(2 * NBUF)]
    ),
)


def _dot(a, b):
    return jnp.dot(a, b, preferred_element_type=_f32)


def _mm_in_body(x_ref, w_ref, b_ref, o_ref):
    o_ref[...] = jax.nn.relu(_dot(x_ref[...], w_ref[...]) + b_ref[...])


def _mm_in(x, w, b, k):
    return pl.pallas_call(
        _mm_in_body,
        grid=(GE,),
        in_specs=[
            pl.BlockSpec((TE, k), lambda i: (i, 0)),
            pl.BlockSpec((k, HP), lambda i: (0, 0)),
            pl.BlockSpec((1, HP), lambda i: (0, 0)),
        ],
        out_specs=pl.BlockSpec((TE, HP), lambda i: (i, 0)),
        out_shape=jax.ShapeDtypeStruct((E, HP), _f32),
    )(x, w, b)


def _revswap(h):
    down = jnp.concatenate([h[1:], h[:1]], axis=0)
    up = jnp.concatenate([h[-1:], h[:-1]], axis=0)
    rid = lax.broadcasted_iota(jnp.int32, h.shape, 0)
    return jnp.where(rid % 2 == 0, down, up)


def _mm_iter1_body(fb_ref, e_ref, wi_ref, bi_ref, w_ref, b_ref, o_ref):
    h0 = jax.nn.relu(_dot(fb_ref[...], wi_ref[...]) + bi_ref[...])
    m = e_ref[...] - _revswap(h0)
    o_ref[...] = jax.nn.relu(h0 + _dot(m, w_ref[...]) + b_ref[...])


def _mm_iter1(fb, exp, wi, bi, w, b):
    return pl.pallas_call(
        _mm_iter1_body,
        grid=(GE,),
        in_specs=[
            pl.BlockSpec((TE, BF), lambda i: (i, 0)),
            pl.BlockSpec((TE, HP), lambda i: (i, 0)),
            pl.BlockSpec((BF, HP), lambda i: (0, 0)),
            pl.BlockSpec((1, HP), lambda i: (0, 0)),
            pl.BlockSpec((HP, HP), lambda i: (0, 0)),
            pl.BlockSpec((1, HP), lambda i: (0, 0)),
        ],
        out_specs=pl.BlockSpec((TE, HP), lambda i: (i, 0)),
        out_shape=jax.ShapeDtypeStruct((E, HP), _f32),
    )(fb, exp, wi, bi, w, b)


def _mm_iter2_body(fb_ref, e_ref, h_ref, wi_ref, bi_ref, w_ref, b_ref,
                   o_ref):
    h0 = jax.nn.relu(_dot(fb_ref[...], wi_ref[...]) + bi_ref[...])
    m = e_ref[...] - _revswap(h_ref[...])
    o_ref[...] = jax.nn.relu(h0 + _dot(m, w_ref[...]) + b_ref[...])


def _mm_iter2(fb, exp, h, wi, bi, w, b):
    return pl.pallas_call(
        _mm_iter2_body,
        grid=(GE,),
        in_specs=[
            pl.BlockSpec((TE, BF), lambda i: (i, 0)),
            pl.BlockSpec((TE, HP), lambda i: (i, 0)),
            pl.BlockSpec((TE, HP), lambda i: (i, 0)),
            pl.BlockSpec((BF, HP), lambda i: (0, 0)),
            pl.BlockSpec((1, HP), lambda i: (0, 0)),
            pl.BlockSpec((HP, HP), lambda i: (0, 0)),
            pl.BlockSpec((1, HP), lambda i: (0, 0)),
        ],
        out_specs=pl.BlockSpec((TE, HP), lambda i: (i, 0)),
        out_shape=jax.ShapeDtypeStruct((E, HP), _f32),
    )(fb, exp, h, wi, bi, w, b)


def _mm_atom_body(fa_ref, a_ref, wf_ref, wa_ref, b_ref, o_ref):
    o_ref[...] = jax.nn.relu(
        _dot(fa_ref[...], wf_ref[...]) + _dot(a_ref[...], wa_ref[...])
        + b_ref[...])


def _mm_atom(fa, amsg, wf, wa, b):
    return pl.pallas_call(
        _mm_atom_body,
        grid=(GA,),
        in_specs=[
            pl.BlockSpec((TA, AF), lambda i: (i, 0)),
            pl.BlockSpec((TA, HP), lambda i: (i, 0)),
            pl.BlockSpec((AF, HP), lambda i: (0, 0)),
            pl.BlockSpec((HP, HP), lambda i: (0, 0)),
            pl.BlockSpec((1, HP), lambda i: (0, 0)),
        ],
        out_specs=pl.BlockSpec((TA, HP), lambda i: (i, 0)),
        out_shape=jax.ShapeDtypeStruct((NA, HP), _f32),
    )(fa, amsg, wf, wa, b)


def _diff_body(r_ref, p_ref, w_ref, b_ref, d_ref, h_ref):
    d = p_ref[...] - r_ref[...]
    d_ref[...] = d
    h_ref[...] = jax.nn.relu(_dot(d, w_ref[...]) + b_ref[...])


def _diff_in(r_h, p_h, w, b):
    return pl.pallas_call(
        _diff_body,
        grid=(GA,),
        in_specs=[
            pl.BlockSpec((TA, HP), lambda i: (i, 0)),
            pl.BlockSpec((TA, HP), lambda i: (i, 0)),
            pl.BlockSpec((HP, HP), lambda i: (0, 0)),
            pl.BlockSpec((1, HP), lambda i: (0, 0)),
        ],
        out_specs=[
            pl.BlockSpec((TA, HP), lambda i: (i, 0)),
            pl.BlockSpec((TA, HP), lambda i: (i, 0)),
        ],
        out_shape=[
            jax.ShapeDtypeStruct((NA, HP), _f32),
            jax.ShapeDtypeStruct((NA, HP), _f32),
        ],
    )(r_h, p_h, w, b)


def _diffiter_body(h0_ref, n0_ref, n1_ref, n2_ref, n3_ref, w_ref, b_ref,
                   o_ref):
    nei = jnp.concatenate(
        [n0_ref[0], n1_ref[0], n2_ref[0], n3_ref[0]], axis=1)
    o_ref[...] = jax.nn.relu(h0_ref[...] + _dot(nei, w_ref[...]) + b_ref[...])


def _diffiter(hd0, nei, w, b):
    nspec = [pl.BlockSpec((1, TA, QW), (lambda i, g=g: (g, i, 0)))
             for g in range(4)]
    return pl.pallas_call(
        _diffiter_body,
        grid=(GA,),
        in_specs=[pl.BlockSpec((TA, HP), lambda i: (i, 0))] + nspec + [
            pl.BlockSpec((4 * QW, HP), lambda i: (0, 0)),
            pl.BlockSpec((1, HP), lambda i: (0, 0)),
        ],
        out_specs=pl.BlockSpec((TA, HP), lambda i: (i, 0)),
        out_shape=jax.ShapeDtypeStruct((NA, HP), _f32),
    )(hd0, nei, nei, nei, nei, w, b)


def _readout_body(d_ref, h_ref, mid_ref, wd_ref, wh_ref, bo_ref,
                  w1_ref, b1_ref, w2_ref, b2_ref, w3_ref, b3_ref,
                  o_ref, acc_ref):
    i = pl.program_id(0)
    ahd = jax.nn.relu(
        _dot(d_ref[...], wd_ref[...]) + _dot(h_ref[...], wh_ref[...])
        + bo_ref[...])
    mids = mid_ref[0, 0, :]
    onehot_t = (lax.broadcasted_iota(jnp.int32, (NM, TA), 0)
                == mids[None, :]).astype(_f32)
    xcat = jnp.concatenate([ahd, jnp.ones((TA, 1), _f32)], axis=1)
    part = _dot(onehot_t, xcat)

    @pl.when(i == 0)
    def _():
        acc_ref[...] = part

    @pl.when(i > 0)
    def _():
        acc_ref[...] = acc_ref[...] + part

    @pl.when(i == GA - 1)
    def _():
        accv = acc_ref[...]
        cnt = jnp.maximum(accv[:, HP:HP + 1], 1.0)
        mv = accv[:, :HP] / cnt
        x = jax.nn.relu(_dot(mv, w1_ref[...]) + b1_ref[...])
        x = jax.nn.relu(_dot(x, w2_ref[...]) + b2_ref[...])
        o_ref[...] = _dot(x, w3_ref[...]) + b3_ref[...]


def _readout(diff, hd, mids3, wd, wh, bo, w1, b1, w2, b2, w3, b3):
    return pl.pallas_call(
        _readout_body,
        grid=(GA,),
        in_specs=[
            pl.BlockSpec((TA, HP), lambda i: (i, 0)),
            pl.BlockSpec((TA, HP), lambda i: (i, 0)),
            pl.BlockSpec((1, 1, TA), lambda i: (i, 0, 0)),
            pl.BlockSpec((HP, HP), lambda i: (0, 0)),
            pl.BlockSpec((HP, HP), lambda i: (0, 0)),
            pl.BlockSpec((1, HP), lambda i: (0, 0)),
            pl.BlockSpec((HP, HP), lambda i: (0, 0)),
            pl.BlockSpec((1, HP), lambda i: (0, 0)),
            pl.BlockSpec((HP, HP), lambda i: (0, 0)),
            pl.BlockSpec((1, HP), lambda i: (0, 0)),
            pl.BlockSpec((HP, TASK), lambda i: (0, 0)),
            pl.BlockSpec((1, TASK), lambda i: (0, 0)),
        ],
        out_specs=pl.BlockSpec((NM, TASK), lambda i: (0, 0)),
        out_shape=jax.ShapeDtypeStruct((NM, TASK), _f32),
        scratch_shapes=[pltpu.VMEM((NM, HP + 1), _f32)],
    )(diff, hd, mids3, wd, wh, bo, w1, b1, w2, b2, w3, b3)


def _padw(w, r, c):
    return jnp.zeros((r, c), _f32).at[:w.shape[0], :w.shape[1]].set(w)


def _mpn_dual(f_atoms_r, f_bonds_r, f_atoms_p, f_bonds_p, src, dst, zpad,
              wi, bi, wh, bh, wf, wa, bo):
    seg_exp_r = _make_seg(True)
    seg_fin_r = _make_seg(False)
    seg_exp_p = _make_seg(True, flip=True)
    seg_fin_p = _make_seg(False, flip=True)
    h0r = _mm_in(f_bonds_r, wi, bi, BF)
    h0p = _mm_in(f_bonds_p, wi, bi, BF)
    e1r = seg_exp_r(h0r, src, dst, zpad)
    e1p = seg_exp_p(h0p, src, dst, zpad)
    h1r = _mm_iter1(f_bonds_r, e1r, wi, bi, wh, bh)
    h1p = _mm_iter1(f_bonds_p, e1p, wi, bi, wh, bh)
    e2r = seg_exp_r(h1r, src, dst, zpad)
    e2p = seg_exp_p(h1p, src, dst, zpad)
    h2r = _mm_iter2(f_bonds_r, e2r, h1r, wi, bi, wh, bh)
    h2p = _mm_iter2(f_bonds_p, e2p, h1p, wi, bi, wh, bh)
    ar = seg_fin_r(h2r, src, dst, zpad)
    ap = seg_fin_p(h2p, src, dst, zpad)
    r_h = _mm_atom(f_atoms_r, ar, wf, wa, bo)
    p_h = _mm_atom(f_atoms_p, ap, wf, wa, bo)
    return r_h, p_h


def kernel(f_atoms_r, f_bonds_r, f_atoms_p, f_bonds_p, W_i, b_i, W_h, b_h,
           W_o, b_o, Wd_i, bd_i, Wd_h, bd_h, Wd_o, bd_o, W1, b1, W2, b2,
           W3, b3, edge_index, rev, mol_ids, gpu):
    src = edge_index[0]
    dst = edge_index[1]

    wi = _padw(W_i, BF, HP)
    bi = _padw(b_i[None], 1, HP)
    wh = _padw(W_h, HP, HP)
    bh = _padw(b_h[None], 1, HP)
    wf = _padw(W_o[:AF], AF, HP)
    wa = _padw(W_o[AF:], HP, HP)
    bo = _padw(b_o[None], 1, HP)
    wdi = _padw(Wd_i, HP, HP)
    bdi = _padw(bd_i[None], 1, HP)
    wdh = _padw(Wd_h, 4 * QW, HP)
    bdh = _padw(bd_h[None], 1, HP)
    wdo_d = _padw(Wd_o[:H], HP, HP)
    wdo_h = _padw(Wd_o[H:], HP, HP)
    bdo = _padw(bd_o[None], 1, HP)
    w1 = _padw(W1, HP, HP)
    b1p = _padw(b1[None], 1, HP)
    w2 = _padw(W2, HP, HP)
    b2p = _padw(b2[None], 1, HP)
    w3 = _padw(W3, HP, TASK)
    b3p = b3[None]

    zpad = jnp.zeros((ZR, GW), _f32)
    r_h, p_h = _mpn_dual(f_atoms_r, f_bonds_r, f_atoms_p, f_bonds_p,
                         src, dst, zpad, wi, bi, wh, bh, wf, wa, bo)

    diff, hd0 = _diff_in(r_h, p_h, wdi, bdi)
    nei1 = _sc_diff(hd0, src, dst)
    hd1 = _diffiter(hd0, nei1, wdh, bdh)
    nei2 = _sc_diff(hd1, src, dst)
    hd2 = _diffiter(hd0, nei2, wdh, bdh)

    mids3 = mol_ids.reshape(GA, 1, TA)
    return _readout(diff, hd2, mids3, wdo_d, wdo_h, bdo,
                    w1, b1p, w2, b2p, w3, b3p)


# speedup vs baseline: 1.0327x; 1.0327x over previous
import functools

import jax
import jax.numpy as jnp
from jax import lax
from jax.experimental import pallas as pl
from jax.experimental.pallas import tpu as pltpu
from jax.experimental.pallas import tpu_sc as plsc

E = 320000
NA = 10000
NM = 512
AF = 133
BF = 147
H = 300
HP = 384
NC = 2
NSUB = 16
GW = 128
NG = HP // GW
QW = 80
CHUNK = 64
NBUF = 4
CPS = (E // CHUNK) // NSUB
CPS -= CPS % NBUF
NXTRA = (E // CHUNK) - NSUB * CPS
RPS = 624
NREM = NA - NSUB * RPS
ZR = 16
STR = 52
TE = 512
GE = E // TE
TA = 1000
GA = NA // TA
TASK = 2

_f32 = jnp.float32


def _zero_zbuf(zbuf, width):
    zv = jnp.zeros((16,), _f32)
    for r in range(ZR):
        for k in range(width // 16):
            zbuf[r, pl.ds(k * 16, 16)] = zv


def _zero_rows(zbuf, acc, s):
    r0 = s * RPS
    for j in range(RPS // ZR):
        pltpu.sync_copy(zbuf, acc.at[pl.ds(r0 + j * ZR, ZR)])

    @pl.when(s == NSUB - 1)
    def _():
        pltpu.sync_copy(zbuf.at[pl.ds(0, NREM)],
                        acc.at[pl.ds(NSUB * RPS, NREM)])


def _sc_seg_body(do_expand, flip, h_hbm, src_hbm, dst_hbm, z_hbm, out_hbm,
                 acc, b0, b1, b2, b3, i0b, i1b, i2b, i3b, zbuf,
                 l0, l1, l2, l3, t0, t1, t2, t3):
    c = lax.axis_index("c")
    if flip:
        c = 1 - c
    s = lax.axis_index("s")
    bufs = (b0, b1, b2, b3)
    idxs = (i0b, i1b, i2b, i3b)
    lsem = (l0, l1, l2, l3)
    tsem = (t0, t1, t2, t3)
    pltpu.sync_copy(z_hbm, zbuf)
    c0 = s * CPS
    nq = CPS // NBUF

    for p in range(2):
        g = c + 2 * p
        active = g < NG
        col0 = pl.multiple_of(g * GW, GW)
        _zero_rows(zbuf, acc, s)
        plsc.subcore_barrier()

        def ld1(i, b):
            pltpu.async_copy(dst_hbm.at[pl.ds(i * CHUNK, CHUNK)],
                             idxs[b], lsem[b])
            pltpu.async_copy(h_hbm.at[pl.ds(i * CHUNK, CHUNK),
                                      pl.ds(col0, GW)],
                             bufs[b], lsem[b])

        def ld1_wait(i, b):
            pltpu.make_async_copy(dst_hbm.at[pl.ds(i * CHUNK, CHUNK)],
                                  idxs[b], lsem[b]).wait()
            pltpu.make_async_copy(h_hbm.at[pl.ds(i * CHUNK, CHUNK),
                                           pl.ds(col0, GW)],
                                  bufs[b], lsem[b]).wait()

        @pl.when(g < NG)
        def _phase1():
            for b in range(NBUF):
                ld1(c0 + b, b)

            def quad1(q, carry):
                for b in range(NBUF):
                    ld1_wait(c0 + q * NBUF + b, b)
                    pltpu.async_copy(bufs[b], acc.at[idxs[b]], tsem[b],
                                     add=True)
                for b in range(NBUF):
                    i = c0 + q * NBUF + b
                    pltpu.make_async_copy(bufs[b], acc.at[idxs[b]],
                                          tsem[b]).wait()

                    @pl.when(q < nq - 1)
                    def _(i=i, b=b):
                        ld1(i + NBUF, b)
                return carry

            lax.fori_loop(0, nq, quad1, 0)

            @pl.when(s < NXTRA)
            def _():
                i = NSUB * CPS + s
                pltpu.sync_copy(dst_hbm.at[pl.ds(i * CHUNK, CHUNK)], i0b)
                pltpu.sync_copy(h_hbm.at[pl.ds(i * CHUNK, CHUNK),
                                         pl.ds(col0, GW)], b0)
                pltpu.sync_copy(b0, acc.at[i0b], add=True)

        plsc.subcore_barrier()

        if do_expand:
            def ld2(i, b):
                pltpu.async_copy(src_hbm.at[pl.ds(i * CHUNK, CHUNK)],
                                 idxs[b], lsem[b])

            def ld2_wait(i, b):
                pltpu.make_async_copy(src_hbm.at[pl.ds(i * CHUNK, CHUNK)],
                                      idxs[b], lsem[b]).wait()

            def st2(i, b):
                return pltpu.async_copy(
                    bufs[b],
                    out_hbm.at[pl.ds(i * CHUNK, CHUNK), pl.ds(col0, GW)],
                    lsem[b])

            def st2_wait(i, b):
                pltpu.make_async_copy(
                    bufs[b],
                    out_hbm.at[pl.ds(i * CHUNK, CHUNK), pl.ds(col0, GW)],
                    lsem[b]).wait()

            @pl.when(g < NG)
            def _phase2():
                for b in range(NBUF):
                    ld2(c0 + b, b)

                def quad2(q, carry):
                    for b in range(NBUF):
                        ld2_wait(c0 + q * NBUF + b, b)
                        pltpu.async_copy(acc.at[idxs[b]], bufs[b], tsem[b])
                    for b in range(NBUF):
                        i = c0 + q * NBUF + b
                        pltpu.make_async_copy(acc.at[idxs[b]], bufs[b],
                                              tsem[b]).wait()
                        st2(i, b)
                    for b in range(NBUF):
                        i = c0 + q * NBUF + b
                        st2_wait(i, b)

                        @pl.when(q < nq - 1)
                        def _(i=i, b=b):
                            ld2(i + NBUF, b)
                    return carry

                lax.fori_loop(0, nq, quad2, 0)

                @pl.when(s < NXTRA)
                def _():
                    i = NSUB * CPS + s
                    pltpu.sync_copy(src_hbm.at[pl.ds(i * CHUNK, CHUNK)], i0b)
                    pltpu.sync_copy(acc.at[i0b], b0)
                    pltpu.sync_copy(b0, out_hbm.at[pl.ds(i * CHUNK, CHUNK),
                                                   pl.ds(col0, GW)])
        else:
            @pl.when(g < NG)
            def _drain():
                r0 = s * RPS
                pltpu.sync_copy(acc.at[pl.ds(r0, RPS)],
                                out_hbm.at[pl.ds(r0, RPS),
                                           pl.ds(col0, GW)])

                @pl.when(s == NSUB - 1)
                def _():
                    pltpu.sync_copy(
                        acc.at[pl.ds(NSUB * RPS, NREM)],
                        out_hbm.at[pl.ds(NSUB * RPS, NREM),
                                   pl.ds(col0, GW)])

        plsc.subcore_barrier()


def _make_seg(do_expand, flip=False):
    out_shape = (E, HP) if do_expand else (NA, HP)
    return pl.kernel(
        functools.partial(_sc_seg_body, do_expand, flip),
        out_type=jax.ShapeDtypeStruct(out_shape, _f32),
        mesh=plsc.VectorSubcoreMesh(core_axis_name="c", subcore_axis_name="s"),
        scratch_types=(
            [pltpu.VMEM_SHARED((NA, GW), _f32)]
            + [pltpu.VMEM((CHUNK, GW), _f32) for _ in range(NBUF)]
            + [pltpu.VMEM((CHUNK,), jnp.int32) for _ in range(NBUF)]
            + [pltpu.VMEM((ZR, GW), _f32)]
            + [pltpu.SemaphoreType.DMA for _ in range(2 * NBUF)]
        ),
    )


def _sc_diff_body(hd_hbm, src_hbm, dst_hbm, out_hbm,
                  stage, acc, b0, b1, b2, b3, is0, is1, is2, is3,
                  id0, id1, id2, id3, zbuf, stbuf,
                  l0, l1, l2, l3, t0, t1, t2, t3):
    c = lax.axis_index("c")
    s = lax.axis_index("s")
    bufs = (b0, b1, b2, b3)
    sidx = (is0, is1, is2, is3)
    didx = (id0, id1, id2, id3)
    lsem = (l0, l1, l2, l3)
    tsem = (t0, t1, t2, t3)
    r0 = s * RPS
    c0 = s * CPS
    nq = CPS // NBUF
    _zero_zbuf(zbuf, QW)
    for p in range(2):
        g = 2 * c + p
        dcol = pl.multiple_of(g * QW, 16)
        for j in range(RPS // STR):
            pltpu.sync_copy(hd_hbm.at[pl.ds(r0 + j * STR, STR),
                                      pl.ds(dcol, QW)], stbuf)
            pltpu.sync_copy(stbuf, stage.at[pl.ds(r0 + j * STR, STR)])

        @pl.when(s == NSUB - 1)
        def _():
            pltpu.sync_copy(hd_hbm.at[pl.ds(NSUB * RPS, NREM),
                                      pl.ds(dcol, QW)],
                            stbuf.at[pl.ds(0, NREM)])
            pltpu.sync_copy(stbuf.at[pl.ds(0, NREM)],
                            stage.at[pl.ds(NSUB * RPS, NREM)])

        _zero_rows(zbuf, acc, s)
        plsc.subcore_barrier()

        def ld(i, b):
            pltpu.async_copy(src_hbm.at[pl.ds(i * CHUNK, CHUNK)],
                             sidx[b], lsem[b])
            pltpu.async_copy(dst_hbm.at[pl.ds(i * CHUNK, CHUNK)],
                             didx[b], lsem[b])

        def ld_wait(i, b):
            pltpu.make_async_copy(src_hbm.at[pl.ds(i * CHUNK, CHUNK)],
                                  sidx[b], lsem[b]).wait()
            pltpu.make_async_copy(dst_hbm.at[pl.ds(i * CHUNK, CHUNK)],
                                  didx[b], lsem[b]).wait()

        for b in range(NBUF):
            ld(c0 + b, b)

        def quad(q, carry):
            for b in range(NBUF):
                i = c0 + q * NBUF + b
                ld_wait(i, b)
                pltpu.async_copy(stage.at[sidx[b]], bufs[b], tsem[b])
            for b in range(NBUF):
                pltpu.make_async_copy(stage.at[sidx[b]], bufs[b],
                                      tsem[b]).wait()
                pltpu.async_copy(bufs[b], acc.at[didx[b]], tsem[b], add=True)
            for b in range(NBUF):
                i = c0 + q * NBUF + b
                pltpu.make_async_copy(bufs[b], acc.at[didx[b]],
                                      tsem[b]).wait()

                @pl.when(q < nq - 1)
                def _(i=i, b=b):
                    ld(i + NBUF, b)
            return carry

        lax.fori_loop(0, nq, quad, 0)

        @pl.when(s < NXTRA)
        def _():
            i = NSUB * CPS + s
            pltpu.sync_copy(src_hbm.at[pl.ds(i * CHUNK, CHUNK)], is0)
            pltpu.sync_copy(dst_hbm.at[pl.ds(i * CHUNK, CHUNK)], id0)
            pltpu.sync_copy(stage.at[is0], b0)
            pltpu.sync_copy(b0, acc.at[id0], add=True)

        plsc.subcore_barrier()
        pltpu.sync_copy(acc.at[pl.ds(r0, RPS)],
                        out_hbm.at[g, pl.ds(r0, RPS)])

        @pl.when(s == NSUB - 1)
        def _():
            pltpu.sync_copy(acc.at[pl.ds(NSUB * RPS, NREM)],
                            out_hbm.at[g, pl.ds(NSUB * RPS, NREM)])

        plsc.subcore_barrier()


_sc_diff = pl.kernel(
    _sc_diff_body,
    out_type=jax.ShapeDtypeStruct((4, NA, QW), _f32),
    mesh=plsc.VectorSubcoreMesh(core_axis_name="c", subcore_axis_name="s"),
    compiler_params=pltpu.CompilerParams(use_tc_tiling_on_sc=False),
    scratch_types=(
        [pltpu.VMEM_SHARED((NA, QW), _f32),
         pltpu.VMEM_SHARED((NA, QW), _f32)]
        + [pltpu.VMEM((CHUNK, QW), _f32) for _ in range(NBUF)]
        + [pltpu.VMEM((CHUNK,), jnp.int32) for _ in range(2 * NBUF)]
        + [pltpu.VMEM((ZR, QW), _f32)]
        + [pltpu.VMEM((STR, QW), _f32)]
        + [pltpu.SemaphoreType.DMA for _ in range(2 * NBUF)]
    ),
)


def _dot(a, b):
    return jnp.dot(a, b, preferred_element_type=_f32)


def _mm_in_body(x_ref, w_ref, b_ref, o_ref):
    o_ref[...] = jax.nn.relu(_dot(x_ref[...], w_ref[...]) + b_ref[...])


def _mm_in(x, w, b, k):
    return pl.pallas_call(
        _mm_in_body,
        grid=(GE,),
        in_specs=[
            pl.BlockSpec((TE, k), lambda i: (i, 0)),
            pl.BlockSpec((k, HP), lambda i: (0, 0)),
            pl.BlockSpec((1, HP), lambda i: (0, 0)),
        ],
        out_specs=pl.BlockSpec((TE, HP), lambda i: (i, 0)),
        out_shape=jax.ShapeDtypeStruct((E, HP), _f32),
    )(x, w, b)


def _revswap(h):
    down = jnp.concatenate([h[1:], h[:1]], axis=0)
    up = jnp.concatenate([h[-1:], h[:-1]], axis=0)
    rid = lax.broadcasted_iota(jnp.int32, h.shape, 0)
    return jnp.where(rid % 2 == 0, down, up)


def _mm_iter1_body(fb_ref, e_ref, wi_ref, bi_ref, w_ref, b_ref, o_ref):
    h0 = jax.nn.relu(_dot(fb_ref[...], wi_ref[...]) + bi_ref[...])
    m = e_ref[...] - _revswap(h0)
    o_ref[...] = jax.nn.relu(h0 + _dot(m, w_ref[...]) + b_ref[...])


def _mm_iter1(fb, exp, wi, bi, w, b):
    return pl.pallas_call(
        _mm_iter1_body,
        grid=(GE,),
        in_specs=[
            pl.BlockSpec((TE, BF), lambda i: (i, 0)),
            pl.BlockSpec((TE, HP), lambda i: (i, 0)),
            pl.BlockSpec((BF, HP), lambda i: (0, 0)),
            pl.BlockSpec((1, HP), lambda i: (0, 0)),
            pl.BlockSpec((HP, HP), lambda i: (0, 0)),
            pl.BlockSpec((1, HP), lambda i: (0, 0)),
        ],
        out_specs=pl.BlockSpec((TE, HP), lambda i: (i, 0)),
        out_shape=jax.ShapeDtypeStruct((E, HP), _f32),
    )(fb, exp, wi, bi, w, b)


def _mm_iter2_body(fb_ref, e_ref, h_ref, wi_ref, bi_ref, w_ref, b_ref,
                   o_ref):
    h0 = jax.nn.relu(_dot(fb_ref[...], wi_ref[...]) + bi_ref[...])
    m = e_ref[...] - _revswap(h_ref[...])
    o_ref[...] = jax.nn.relu(h0 + _dot(m, w_ref[...]) + b_ref[...])


def _mm_iter2(fb, exp, h, wi, bi, w, b):
    return pl.pallas_call(
        _mm_iter2_body,
        grid=(GE,),
        in_specs=[
            pl.BlockSpec((TE, BF), lambda i: (i, 0)),
            pl.BlockSpec((TE, HP), lambda i: (i, 0)),
            pl.BlockSpec((TE, HP), lambda i: (i, 0)),
            pl.BlockSpec((BF, HP), lambda i: (0, 0)),
            pl.BlockSpec((1, HP), lambda i: (0, 0)),
            pl.BlockSpec((HP, HP), lambda i: (0, 0)),
            pl.BlockSpec((1, HP), lambda i: (0, 0)),
        ],
        out_specs=pl.BlockSpec((TE, HP), lambda i: (i, 0)),
        out_shape=jax.ShapeDtypeStruct((E, HP), _f32),
    )(fb, exp, h, wi, bi, w, b)


def _mm_atom_body(fa_ref, a_ref, wf_ref, wa_ref, b_ref, o_ref):
    o_ref[...] = jax.nn.relu(
        _dot(fa_ref[...], wf_ref[...]) + _dot(a_ref[...], wa_ref[...])
        + b_ref[...])


def _mm_atom(fa, amsg, wf, wa, b):
    return pl.pallas_call(
        _mm_atom_body,
        grid=(GA,),
        in_specs=[
            pl.BlockSpec((TA, AF), lambda i: (i, 0)),
            pl.BlockSpec((TA, HP), lambda i: (i, 0)),
            pl.BlockSpec((AF, HP), lambda i: (0, 0)),
            pl.BlockSpec((HP, HP), lambda i: (0, 0)),
            pl.BlockSpec((1, HP), lambda i: (0, 0)),
        ],
        out_specs=pl.BlockSpec((TA, HP), lambda i: (i, 0)),
        out_shape=jax.ShapeDtypeStruct((NA, HP), _f32),
    )(fa, amsg, wf, wa, b)


def _diff_body(r_ref, p_ref, w_ref, b_ref, d_ref, h_ref):
    d = p_ref[...] - r_ref[...]
    d_ref[...] = d
    h_ref[...] = jax.nn.relu(_dot(d, w_ref[...]) + b_ref[...])


def _diff_in(r_h, p_h, w, b):
    return pl.pallas_call(
        _diff_body,
        grid=(GA,),
        in_specs=[
            pl.BlockSpec((TA, HP), lambda i: (i, 0)),
            pl.BlockSpec((TA, HP), lambda i: (i, 0)),
            pl.BlockSpec((HP, HP), lambda i: (0, 0)),
            pl.BlockSpec((1, HP), lambda i: (0, 0)),
        ],
        out_specs=[
            pl.BlockSpec((TA, HP), lambda i: (i, 0)),
            pl.BlockSpec((TA, HP), lambda i: (i, 0)),
        ],
        out_shape=[
            jax.ShapeDtypeStruct((NA, HP), _f32),
            jax.ShapeDtypeStruct((NA, HP), _f32),
        ],
    )(r_h, p_h, w, b)


def _diffiter_body(h0_ref, n0_ref, n1_ref, n2_ref, n3_ref, w_ref, b_ref,
                   o_ref):
    nei = jnp.concatenate(
        [n0_ref[0], n1_ref[0], n2_ref[0], n3_ref[0]], axis=1)
    o_ref[...] = jax.nn.relu(h0_ref[...] + _dot(nei, w_ref[...]) + b_ref[...])


def _diffiter(hd0, nei, w, b):
    nspec = [pl.BlockSpec((1, TA, QW), (lambda i, g=g: (g, i, 0)))
             for g in range(4)]
    return pl.pallas_call(
        _diffiter_body,
        grid=(GA,),
        in_specs=[pl.BlockSpec((TA, HP), lambda i: (i, 0))] + nspec + [
            pl.BlockSpec((4 * QW, HP), lambda i: (0, 0)),
            pl.BlockSpec((1, HP), lambda i: (0, 0)),
        ],
        out_specs=pl.BlockSpec((TA, HP), lambda i: (i, 0)),
        out_shape=jax.ShapeDtypeStruct((NA, HP), _f32),
    )(hd0, nei, nei, nei, nei, w, b)


def _readout_body(d_ref, h_ref, mid_ref, wd_ref, wh_ref, bo_ref,
                  w1_ref, b1_ref, w2_ref, b2_ref, w3_ref, b3_ref,
                  o_ref, acc_ref):
    i = pl.program_id(0)
    ahd = jax.nn.relu(
        _dot(d_ref[...], wd_ref[...]) + _dot(h_ref[...], wh_ref[...])
        + bo_ref[...])
    mids = mid_ref[0, 0, :]
    onehot_t = (lax.broadcasted_iota(jnp.int32, (NM, TA), 0)
                == mids[None, :]).astype(_f32)
    xcat = jnp.concatenate([ahd, jnp.ones((TA, 1), _f32)], axis=1)
    part = _dot(onehot_t, xcat)

    @pl.when(i == 0)
    def _():
        acc_ref[...] = part

    @pl.when(i > 0)
    def _():
        acc_ref[...] = acc_ref[...] + part

    @pl.when(i == GA - 1)
    def _():
        accv = acc_ref[...]
        cnt = jnp.maximum(accv[:, HP:HP + 1], 1.0)
        mv = accv[:, :HP] / cnt
        x = jax.nn.relu(_dot(mv, w1_ref[...]) + b1_ref[...])
        x = jax.nn.relu(_dot(x, w2_ref[...]) + b2_ref[...])
        o_ref[...] = _dot(x, w3_ref[...]) + b3_ref[...]


def _readout(diff, hd, mids3, wd, wh, bo, w1, b1, w2, b2, w3, b3):
    return pl.pallas_call(
        _readout_body,
        grid=(GA,),
        in_specs=[
            pl.BlockSpec((TA, HP), lambda i: (i, 0)),
            pl.BlockSpec((TA, HP), lambda i: (i, 0)),
            pl.BlockSpec((1, 1, TA), lambda i: (i, 0, 0)),
            pl.BlockSpec((HP, HP), lambda i: (0, 0)),
            pl.BlockSpec((HP, HP), lambda i: (0, 0)),
            pl.BlockSpec((1, HP), lambda i: (0, 0)),
            pl.BlockSpec((HP, HP), lambda i: (0, 0)),
            pl.BlockSpec((1, HP), lambda i: (0, 0)),
            pl.BlockSpec((HP, HP), lambda i: (0, 0)),
            pl.BlockSpec((1, HP), lambda i: (0, 0)),
            pl.BlockSpec((HP, TASK), lambda i: (0, 0)),
            pl.BlockSpec((1, TASK), lambda i: (0, 0)),
        ],
        out_specs=pl.BlockSpec((NM, TASK), lambda i: (0, 0)),
        out_shape=jax.ShapeDtypeStruct((NM, TASK), _f32),
        scratch_shapes=[pltpu.VMEM((NM, HP + 1), _f32)],
    )(diff, hd, mids3, wd, wh, bo, w1, b1, w2, b2, w3, b3)


def _padw(w, r, c):
    return jnp.zeros((r, c), _f32).at[:w.shape[0], :w.shape[1]].set(w)


def _mpn_dual(f_atoms_r, f_bonds_r, f_atoms_p, f_bonds_p, src, dst, zpad,
              wi, bi, wh, bh, wf, wa, bo):
    seg_exp_r = _make_seg(True)
    seg_fin_r = _make_seg(False)
    seg_exp_p = _make_seg(True, flip=True)
    seg_fin_p = _make_seg(False, flip=True)
    h0r = _mm_in(f_bonds_r, wi, bi, BF)
    h0p = _mm_in(f_bonds_p, wi, bi, BF)
    e1r = seg_exp_r(h0r, src, dst, zpad)
    e1p = seg_exp_p(h0p, src, dst, zpad)
    h1r = _mm_iter1(f_bonds_r, e1r, wi, bi, wh, bh)
    h1p = _mm_iter1(f_bonds_p, e1p, wi, bi, wh, bh)
    e2r = seg_exp_r(h1r, src, dst, zpad)
    e2p = seg_exp_p(h1p, src, dst, zpad)
    h2r = _mm_iter2(f_bonds_r, e2r, h1r, wi, bi, wh, bh)
    h2p = _mm_iter2(f_bonds_p, e2p, h1p, wi, bi, wh, bh)
    ar = seg_fin_r(h2r, src, dst, zpad)
    ap = seg_fin_p(h2p, src, dst, zpad)
    r_h = _mm_atom(f_atoms_r, ar, wf, wa, bo)
    p_h = _mm_atom(f_atoms_p, ap, wf, wa, bo)
    return r_h, p_h


def kernel(f_atoms_r, f_bonds_r, f_atoms_p, f_bonds_p, W_i, b_i, W_h, b_h,
           W_o, b_o, Wd_i, bd_i, Wd_h, bd_h, Wd_o, bd_o, W1, b1, W2, b2,
           W3, b3, edge_index, rev, mol_ids, gpu):
    src = edge_index[0]
    dst = edge_index[1]

    wi = _padw(W_i, BF, HP)
    bi = _padw(b_i[None], 1, HP)
    wh = _padw(W_h, HP, HP)
    bh = _padw(b_h[None], 1, HP)
    wf = _padw(W_o[:AF], AF, HP)
    wa = _padw(W_o[AF:], HP, HP)
    bo = _padw(b_o[None], 1, HP)
    wdi = _padw(Wd_i, HP, HP)
    bdi = _padw(bd_i[None], 1, HP)
    wdh = _padw(Wd_h, 4 * QW, HP)
    bdh = _padw(bd_h[None], 1, HP)
    wdo_d = _padw(Wd_o[:H], HP, HP)
    wdo_h = _padw(Wd_o[H:], HP, HP)
    bdo = _padw(bd_o[None], 1, HP)
    w1 = _padw(W1, HP, HP)
    b1p = _padw(b1[None], 1, HP)
    w2 = _padw(W2, HP, HP)
    b2p = _padw(b2[None], 1, HP)
    w3 = _padw(W3, HP, TASK)
    b3p = b3[None]

    zpad = jnp.zeros((ZR, GW), _f32)
    r_h, p_h = _mpn_dual(f_atoms_r, f_bonds_r, f_atoms_p, f_bonds_p,
                         src, dst, zpad, wi, bi, wh, bh, wf, wa, bo)

    diff, hd0 = _diff_in(r_h, p_h, wdi, bdi)
    nei1 = _sc_diff(hd0, src, dst)
    hd1 = _diffiter(hd0, nei1, wdh, bdh)
    nei2 = _sc_diff(hd1, src, dst)
    hd2 = _diffiter(hd0, nei2, wdh, bdh)

    mids3 = mol_ids.reshape(GA, 1, TA)
    return _readout(diff, hd2, mids3, wdo_d, wdo_h, bdo,
                    w1, b1p, w2, b2p, w3, b3p)
